# Initial kernel scaffold; baseline (speedup 1.0000x reference)
import jax
import jax.numpy as jnp
from jax.experimental import pallas as pl


def kernel(bert_representation, history_attention_input, mtl_input, slice_mask, slice_num, W, b):
    raise NotImplementedError("write your pallas kernel here")


# speedup vs baseline: 1.8432x; 1.8432x over previous
import functools

import jax
import jax.numpy as jnp
from jax import lax
from jax.experimental import pallas as pl
from jax.experimental.pallas import tpu as pltpu

_T = 11


def _scale_kernel(slice_ref, num_ref, b_ref, hist_ref, mtl_ref, wt_ref,
                  bert_ref, nbert_ref, nmtl_ref, probs_ref):
    i = pl.program_id(0)
    h = hist_ref[0, 0, :]
    w = wt_ref[0, :]
    bias = b_ref[0]
    logit = jnp.sum(h * w) + bias
    length = slice_ref[i]
    t = lax.broadcasted_iota(jnp.int32, (1, _T), 1)
    mask = (t >= _T - length).astype(jnp.float32)
    row = (i < num_ref[0]).astype(jnp.float32)
    logits_row = jnp.where(t == _T - 1, logit, bias)
    e = jnp.exp(logits_row) * mask * row
    p = e / jnp.sum(e)
    probs_ref[0, 0, :] = p[0, :]
    scale = p[0, _T - 1]
    nmtl_ref[0, 0, :] = mtl_ref[0, 0, :] * scale
    nbert_ref[0, :, :] = bert_ref[0, :, :] * scale


def kernel(bert_representation, history_attention_input, mtl_input,
           slice_mask, slice_num, W, b):
    bs, seq, hid = bert_representation.shape
    wt = W.reshape(1, hid)
    num = jnp.asarray(slice_num, jnp.int32).reshape(1)
    grid = (bs,)
    nbert, nmtl, probs3 = pl.pallas_call(
        _scale_kernel,
        grid=grid,
        in_specs=[
            pl.BlockSpec(memory_space=pltpu.SMEM),
            pl.BlockSpec(memory_space=pltpu.SMEM),
            pl.BlockSpec(memory_space=pltpu.SMEM),
            pl.BlockSpec((1, 1, hid), lambda i: (i, 0, 0)),
            pl.BlockSpec((1, 1, hid), lambda i: (i, 0, 0)),
            pl.BlockSpec((1, hid), lambda i: (0, 0)),
            pl.BlockSpec((1, seq, hid), lambda i: (i, 0, 0)),
        ],
        out_specs=[
            pl.BlockSpec((1, seq, hid), lambda i: (i, 0, 0)),
            pl.BlockSpec((1, 1, hid), lambda i: (i, 0, 0)),
            pl.BlockSpec((1, 1, _T), lambda i: (i, 0, 0)),
        ],
        out_shape=[
            jax.ShapeDtypeStruct((bs, seq, hid), jnp.float32),
            jax.ShapeDtypeStruct((bs, 1, hid), jnp.float32),
            jax.ShapeDtypeStruct((bs, 1, _T), jnp.float32),
        ],
        compiler_params=pltpu.CompilerParams(
            dimension_semantics=("arbitrary",),
        ),
    )(slice_mask.astype(jnp.int32), num, b,
      history_attention_input.reshape(bs, 1, hid),
      mtl_input.reshape(bs, 1, hid), wt, bert_representation)
    return nbert, nmtl.reshape(bs, hid), probs3.reshape(bs, _T)


# speedup vs baseline: 2.6565x; 1.4412x over previous
import jax
import jax.numpy as jnp
from jax import lax
from jax.experimental import pallas as pl
from jax.experimental.pallas import tpu as pltpu

_T = 11
_EX = 2
_SQ = 512


def _scale_kernel(num_ref, b_ref, sm_ref, hist_ref, mtl_ref, wt_ref,
                  bert_ref, nbert_ref, nmtl_ref, probs_ref):
    i = pl.program_id(0)
    bs = hist_ref.shape[1]
    w = wt_ref[0, :]
    h = hist_ref[0]
    bias = b_ref[0]
    logit = jnp.sum(h * w[None, :], axis=1) + bias
    t = lax.broadcasted_iota(jnp.int32, (bs, _T), 1)
    r = lax.broadcasted_iota(jnp.int32, (bs, _T), 0)
    lengths = sm_ref[0][:, None]
    mask = (t >= _T - lengths).astype(jnp.float32)
    rowm = (r < num_ref[0]).astype(jnp.float32)
    lrow = jnp.where(t == _T - 1, logit[:, None], bias)
    e = jnp.exp(lrow) * mask * rowm
    p = e / jnp.sum(e, axis=1, keepdims=True)
    s = p[:, _T - 1]
    probs_ref[0] = p
    nmtl_ref[0] = mtl_ref[0] * s[:, None]
    col = lax.broadcasted_iota(jnp.int32, (_EX, bs), 1)
    row = lax.broadcasted_iota(jnp.int32, (_EX, bs), 0)
    sel = (col == i * _EX + row).astype(jnp.float32)
    sblk = jnp.sum(sel * s[None, :], axis=1)
    nbert_ref[...] = bert_ref[...] * sblk[:, None, None]


def kernel(bert_representation, history_attention_input, mtl_input,
           slice_mask, slice_num, W, b):
    bs, seq, hid = bert_representation.shape
    wt = W.reshape(1, hid)
    num = jnp.asarray(slice_num, jnp.int32).reshape(1)
    grid = (bs // _EX, seq // _SQ)
    nbert, nmtl, probs = pl.pallas_call(
        _scale_kernel,
        grid=grid,
        in_specs=[
            pl.BlockSpec(memory_space=pltpu.SMEM),
            pl.BlockSpec(memory_space=pltpu.SMEM),
            pl.BlockSpec((1, bs), lambda i, j: (0, 0)),
            pl.BlockSpec((1, bs, hid), lambda i, j: (0, 0, 0)),
            pl.BlockSpec((1, bs, hid), lambda i, j: (0, 0, 0)),
            pl.BlockSpec((1, hid), lambda i, j: (0, 0)),
            pl.BlockSpec((_EX, _SQ, hid), lambda i, j: (i, j, 0)),
        ],
        out_specs=[
            pl.BlockSpec((_EX, _SQ, hid), lambda i, j: (i, j, 0)),
            pl.BlockSpec((1, bs, hid), lambda i, j: (0, 0, 0)),
            pl.BlockSpec((1, bs, _T), lambda i, j: (0, 0, 0)),
        ],
        out_shape=[
            jax.ShapeDtypeStruct((bs, seq, hid), jnp.float32),
            jax.ShapeDtypeStruct((1, bs, hid), jnp.float32),
            jax.ShapeDtypeStruct((1, bs, _T), jnp.float32),
        ],
        compiler_params=pltpu.CompilerParams(
            dimension_semantics=("arbitrary", "arbitrary"),
        ),
    )(num, b, slice_mask.astype(jnp.int32).reshape(1, bs),
      history_attention_input.reshape(1, bs, hid),
      mtl_input.reshape(1, bs, hid), wt, bert_representation)
    return nbert, nmtl.reshape(bs, hid), probs.reshape(bs, _T)


# speedup vs baseline: 2.8668x; 1.0792x over previous
import jax
import jax.numpy as jnp
from jax import lax
from jax.experimental import pallas as pl
from jax.experimental.pallas import tpu as pltpu

_T = 11
_EX = 4
_SQ = 512


def _scale_kernel(num_ref, b_ref, sm_ref, hist_ref, mtl_ref, wt_ref,
                  bert_ref, nbert_ref, nmtl_ref, probs_ref):
    i = pl.program_id(0)
    bs = hist_ref.shape[1]
    w = wt_ref[0, :]
    h = hist_ref[0]
    bias = b_ref[0]
    logit = jnp.sum(h * w[None, :], axis=1) + bias
    t = lax.broadcasted_iota(jnp.int32, (bs, _T), 1)
    r = lax.broadcasted_iota(jnp.int32, (bs, _T), 0)
    lengths = sm_ref[0][:, None]
    mask = (t >= _T - lengths).astype(jnp.float32)
    rowm = (r < num_ref[0]).astype(jnp.float32)
    lrow = jnp.where(t == _T - 1, logit[:, None], bias)
    e = jnp.exp(lrow) * mask * rowm
    p = e / jnp.sum(e, axis=1, keepdims=True)
    s = p[:, _T - 1]
    probs_ref[0] = p
    nmtl_ref[0] = mtl_ref[0] * s[:, None]
    col = lax.broadcasted_iota(jnp.int32, (_EX, bs), 1)
    row = lax.broadcasted_iota(jnp.int32, (_EX, bs), 0)
    sel = (col == i * _EX + row).astype(jnp.float32)
    sblk = jnp.sum(sel * s[None, :], axis=1)
    nbert_ref[...] = bert_ref[...] * sblk[:, None, None]


def kernel(bert_representation, history_attention_input, mtl_input,
           slice_mask, slice_num, W, b):
    bs, seq, hid = bert_representation.shape
    wt = W.reshape(1, hid)
    num = jnp.asarray(slice_num, jnp.int32).reshape(1)
    grid = (bs // _EX, seq // _SQ)
    nbert, nmtl, probs = pl.pallas_call(
        _scale_kernel,
        grid=grid,
        in_specs=[
            pl.BlockSpec(memory_space=pltpu.SMEM),
            pl.BlockSpec(memory_space=pltpu.SMEM),
            pl.BlockSpec((1, bs), lambda i, j: (0, 0)),
            pl.BlockSpec((1, bs, hid), lambda i, j: (0, 0, 0)),
            pl.BlockSpec((1, bs, hid), lambda i, j: (0, 0, 0)),
            pl.BlockSpec((1, hid), lambda i, j: (0, 0)),
            pl.BlockSpec((_EX, _SQ, hid), lambda i, j: (i, j, 0)),
        ],
        out_specs=[
            pl.BlockSpec((_EX, _SQ, hid), lambda i, j: (i, j, 0)),
            pl.BlockSpec((1, bs, hid), lambda i, j: (0, 0, 0)),
            pl.BlockSpec((1, bs, _T), lambda i, j: (0, 0, 0)),
        ],
        out_shape=[
            jax.ShapeDtypeStruct((bs, seq, hid), jnp.float32),
            jax.ShapeDtypeStruct((1, bs, hid), jnp.float32),
            jax.ShapeDtypeStruct((1, bs, _T), jnp.float32),
        ],
        compiler_params=pltpu.CompilerParams(
            dimension_semantics=("arbitrary", "arbitrary"),
        ),
    )(num, b, slice_mask.astype(jnp.int32).reshape(1, bs),
      history_attention_input.reshape(1, bs, hid),
      mtl_input.reshape(1, bs, hid), wt, bert_representation)
    return nbert, nmtl.reshape(bs, hid), probs.reshape(bs, _T)


# speedup vs baseline: 3.0996x; 1.0812x over previous
import jax
import jax.numpy as jnp
from jax import lax
from jax.experimental import pallas as pl
from jax.experimental.pallas import tpu as pltpu

_T = 11
_EX = 8
_SQ = 512


def _scale_kernel(num_ref, b_ref, sm_ref, hist_ref, mtl_ref, wt_ref,
                  bert_ref, nbert_ref, nmtl_ref, probs_ref):
    i = pl.program_id(0)
    bs = hist_ref.shape[1]
    w = wt_ref[0, :]
    h = hist_ref[0]
    bias = b_ref[0]
    logit = jnp.sum(h * w[None, :], axis=1) + bias
    t = lax.broadcasted_iota(jnp.int32, (bs, _T), 1)
    r = lax.broadcasted_iota(jnp.int32, (bs, _T), 0)
    lengths = sm_ref[0][:, None]
    mask = (t >= _T - lengths).astype(jnp.float32)
    rowm = (r < num_ref[0]).astype(jnp.float32)
    lrow = jnp.where(t == _T - 1, logit[:, None], bias)
    e = jnp.exp(lrow) * mask * rowm
    p = e / jnp.sum(e, axis=1, keepdims=True)
    s = p[:, _T - 1]
    probs_ref[0] = p
    nmtl_ref[0] = mtl_ref[0] * s[:, None]
    col = lax.broadcasted_iota(jnp.int32, (_EX, bs), 1)
    row = lax.broadcasted_iota(jnp.int32, (_EX, bs), 0)
    sel = (col == i * _EX + row).astype(jnp.float32)
    sblk = jnp.sum(sel * s[None, :], axis=1)
    nbert_ref[...] = bert_ref[...] * sblk[:, None, None]


def kernel(bert_representation, history_attention_input, mtl_input,
           slice_mask, slice_num, W, b):
    bs, seq, hid = bert_representation.shape
    wt = W.reshape(1, hid)
    num = jnp.asarray(slice_num, jnp.int32).reshape(1)
    grid = (bs // _EX, seq // _SQ)
    nbert, nmtl, probs = pl.pallas_call(
        _scale_kernel,
        grid=grid,
        in_specs=[
            pl.BlockSpec(memory_space=pltpu.SMEM),
            pl.BlockSpec(memory_space=pltpu.SMEM),
            pl.BlockSpec((1, bs), lambda i, j: (0, 0)),
            pl.BlockSpec((1, bs, hid), lambda i, j: (0, 0, 0)),
            pl.BlockSpec((1, bs, hid), lambda i, j: (0, 0, 0)),
            pl.BlockSpec((1, hid), lambda i, j: (0, 0)),
            pl.BlockSpec((_EX, _SQ, hid), lambda i, j: (i, j, 0)),
        ],
        out_specs=[
            pl.BlockSpec((_EX, _SQ, hid), lambda i, j: (i, j, 0)),
            pl.BlockSpec((1, bs, hid), lambda i, j: (0, 0, 0)),
            pl.BlockSpec((1, bs, _T), lambda i, j: (0, 0, 0)),
        ],
        out_shape=[
            jax.ShapeDtypeStruct((bs, seq, hid), jnp.float32),
            jax.ShapeDtypeStruct((1, bs, hid), jnp.float32),
            jax.ShapeDtypeStruct((1, bs, _T), jnp.float32),
        ],
        compiler_params=pltpu.CompilerParams(
            dimension_semantics=("arbitrary", "arbitrary"),
        ),
    )(num, b, slice_mask.astype(jnp.int32).reshape(1, bs),
      history_attention_input.reshape(1, bs, hid),
      mtl_input.reshape(1, bs, hid), wt, bert_representation)
    return nbert, nmtl.reshape(bs, hid), probs.reshape(bs, _T)
